# baseline (speedup 1.0000x reference)
import functools

import jax
import jax.numpy as jnp
from jax import lax
from jax.experimental import pallas as pl
from jax.experimental.pallas import tpu as pltpu
from jax.experimental.pallas import tpu_sc as plsc

N = 10000
NPAD = 10240
E = 320000
IN = 128
HID = 256
OUT = 128

NC = 2
NS = 16
NW = NC * NS
C = 96
CHPT = 105
EPT = CHPT * C
EPAD = NW * EPT
ACCR = 10112
NPR = ACCR - N
RPT = NPAD // NS
APT = ACCR // NS
AWB = [(i * 96, 96) for i in range(6)] + [(576, 56)]
WB = 96

_mesh = plsc.VectorSubcoreMesh(core_axis_name="c", subcore_axis_name="s")


@functools.partial(
    pl.kernel,
    out_type=jax.ShapeDtypeStruct((NC, NPAD), jnp.float32),
    mesh=_mesh,
    scratch_types=[
        [pltpu.VMEM((C,), jnp.int32)] * 4,
        pltpu.VMEM((C,), jnp.float32),
        pltpu.VMEM((RPT,), jnp.float32),
        pltpu.VMEM_SHARED((NPAD,), jnp.float32),
        [pltpu.SemaphoreType.DMA] * 4,
        [pltpu.SemaphoreType.DMA] * 4,
    ],
)
def _deg_kernel(dst_hbm, ones_hbm, zeros_hbm, out_hbm,
                idx, ones_v, stage_v, acc, semi, sems):
    c = lax.axis_index("c")
    s = lax.axis_index("s")
    pltpu.sync_copy(ones_hbm, ones_v)
    pltpu.sync_copy(zeros_hbm, stage_v)
    pltpu.sync_copy(stage_v, acc.at[pl.ds(s * RPT, RPT)])
    plsc.subcore_barrier()

    ebase = (c * NS + s) * EPT

    def d_at(j):
        return dst_hbm.at[pl.ds(pl.multiple_of(ebase + j * C, 8), C)]

    def load(j, b):
        pltpu.async_copy(d_at(j), idx[b], semi[b])

    def wait_scatter(b):
        pltpu.make_async_copy(ones_v, acc.at[idx[b]], sems[b]).wait()

    def step(j, b, do_load, do_wait):
        if do_load:
            if do_wait:
                wait_scatter((b + 2) % 4)
            load(j + 2, (b + 2) % 4)
        pltpu.make_async_copy(d_at(j), idx[b], semi[b]).wait()
        pltpu.async_copy(ones_v, acc.at[idx[b]], sems[b], add=True)

    load(0, 0)
    load(1, 1)

    def body(k, carry):
        for r in range(4):
            step(4 * k + 2 + r, (2 + r) % 4, True, True)
        return carry

    KMAIN = (CHPT - 4) // 4
    step(0, 0, True, False)
    step(1, 1, True, False)
    lax.fori_loop(0, KMAIN, body, 0)
    for j in range(4 * KMAIN + 2, CHPT):
        step(j, j % 4, j + 2 < CHPT, j - 2 >= 0)
    for j in range(max(0, CHPT - 4), CHPT):
        wait_scatter(j % 4)

    plsc.subcore_barrier()
    pltpu.sync_copy(acc.at[pl.ds(s * RPT, RPT)], stage_v)
    pltpu.sync_copy(stage_v, out_hbm.at[c, pl.ds(s * RPT, RPT)])


@functools.partial(
    pl.kernel,
    out_type=jax.ShapeDtypeStruct((NC, NPAD, IN), jnp.float32),
    mesh=_mesh,
    scratch_types=[
        [pltpu.VMEM((C,), jnp.int32)] * 4,
        [pltpu.VMEM((C,), jnp.int32)] * 4,
        [pltpu.VMEM((C, IN), jnp.float32)] * 4,
        pltpu.VMEM_SHARED((ACCR, IN), jnp.float32),
        [pltpu.SemaphoreType.DMA] * 4,
        [pltpu.SemaphoreType.DMA] * 4,
        [pltpu.SemaphoreType.DMA] * 4,
    ],
)
def _agg_kernel(src_hbm, dst_hbm, tab_hbm, zrows_hbm, out_hbm,
                srcv, dstv, rows, acc, semi, semg, sems):
    c = lax.axis_index("c")
    s = lax.axis_index("s")

    pltpu.sync_copy(zrows_hbm, rows[0])
    for r0, w in AWB:
        pltpu.sync_copy(rows[0].at[pl.ds(0, w)],
                        acc.at[pl.ds(s * APT + r0, w)])
    plsc.subcore_barrier()

    ebase = (c * NS + s) * EPT

    def s_at(j):
        return src_hbm.at[pl.ds(pl.multiple_of(ebase + j * C, 8), C)]

    def d_at(j):
        return dst_hbm.at[pl.ds(pl.multiple_of(ebase + j * C, 8), C)]

    def load_idx(j, b):
        pltpu.async_copy(s_at(j), srcv[b], semi[b])
        pltpu.async_copy(d_at(j), dstv[b], semi[b])

    def start_gather(j, b):
        pltpu.make_async_copy(s_at(j), srcv[b], semi[b]).wait()
        pltpu.make_async_copy(d_at(j), dstv[b], semi[b]).wait()
        pltpu.async_copy(tab_hbm.at[srcv[b]], rows[b], semg[b])

    def start_scatter(b):
        pltpu.make_async_copy(tab_hbm.at[srcv[b]], rows[b], semg[b]).wait()
        pltpu.async_copy(rows[b], acc.at[dstv[b]], sems[b], add=True)

    def wait_scatter(b):
        pltpu.make_async_copy(rows[b], acc.at[dstv[b]], sems[b]).wait()

    def step(j, b, do_load, do_wait, do_gather):
        if do_load:
            if do_wait:
                wait_scatter((b + 2) % 4)
            load_idx(j + 2, (b + 2) % 4)
        if do_gather:
            start_gather(j + 1, (b + 1) % 4)
        start_scatter(b)

    load_idx(0, 0)
    load_idx(1, 1)
    start_gather(0, 0)

    def body(k, carry):
        for r in range(4):
            step(4 * k + 2 + r, (2 + r) % 4, True, True, True)
        return carry

    KMAIN = (CHPT - 4) // 4
    step(0, 0, True, False, True)
    step(1, 1, True, False, True)
    lax.fori_loop(0, KMAIN, body, 0)
    for j in range(4 * KMAIN + 2, CHPT):
        step(j, j % 4, j + 2 < CHPT, j - 2 >= 0, j + 1 < CHPT)
    for j in range(max(0, CHPT - 4), CHPT):
        wait_scatter(j % 4)

    plsc.subcore_barrier()
    for r0, w in AWB:
        a0 = s * APT + r0
        pltpu.sync_copy(acc.at[pl.ds(a0, w)], rows[0].at[pl.ds(0, w)])
        pltpu.sync_copy(rows[0].at[pl.ds(0, w)], out_hbm.at[c, pl.ds(a0, w)])


RB = 1024
GRID = NPAD // RB


def _dis_block(pt):
    return lax.rsqrt(1.0 + pt[:, 0:1] + pt[:, 1:2])


def _xs_body(pt_ref, x_ref, xs_ref):
    dis = _dis_block(pt_ref[...])
    xs_ref[...] = x_ref[...] * dis


def _mid_body(pt_ref, x_ref, s1_ref, W1_ref, b1_ref, W2_ref, y_ref, ys_ref):
    dis = _dis_block(pt_ref[...])
    agg = s1_ref[0] + s1_ref[1]
    u = dis * agg + (dis * dis) * x_ref[...]
    h = jnp.dot(u, W1_ref[...], preferred_element_type=jnp.float32) + b1_ref[...]
    h = jnp.maximum(h, 0.0)
    y = jnp.dot(h, W2_ref[...], preferred_element_type=jnp.float32)
    y_ref[...] = y
    ys_ref[...] = y * dis


def _out_body(pt_ref, y_ref, s2_ref, b2_ref, Wl_ref, bl_ref, o_ref):
    dis = _dis_block(pt_ref[...])
    agg = s2_ref[0] + s2_ref[1]
    h2 = jnp.maximum(dis * agg + (dis * dis) * y_ref[...] + b2_ref[...], 0.0)
    o_ref[...] = jnp.dot(h2, Wl_ref[...], preferred_element_type=jnp.float32) + bl_ref[...]


def _row_spec(width):
    return pl.BlockSpec((RB, width), lambda i: (i, 0))


def _part_spec(width):
    return pl.BlockSpec((NC, RB, width), lambda i: (0, i, 0))


def _full_spec(shape):
    return pl.BlockSpec(shape, lambda i: tuple(0 for _ in shape))


_xs_call = pl.pallas_call(
    _xs_body,
    grid=(GRID,),
    in_specs=[_row_spec(2), _row_spec(IN)],
    out_specs=_row_spec(IN),
    out_shape=jax.ShapeDtypeStruct((NPAD, IN), jnp.float32),
)

_mid_call = pl.pallas_call(
    _mid_body,
    grid=(GRID,),
    in_specs=[
        _row_spec(2),
        _row_spec(IN),
        _part_spec(IN),
        _full_spec((IN, HID)),
        _full_spec((1, HID)),
        _full_spec((HID, OUT)),
    ],
    out_specs=[_row_spec(OUT), _row_spec(OUT)],
    out_shape=[
        jax.ShapeDtypeStruct((NPAD, OUT), jnp.float32),
        jax.ShapeDtypeStruct((NPAD, OUT), jnp.float32),
    ],
)

_out_call = pl.pallas_call(
    _out_body,
    grid=(GRID,),
    in_specs=[
        _row_spec(2),
        _row_spec(OUT),
        _part_spec(OUT),
        _full_spec((1, OUT)),
        _full_spec((OUT, 128)),
        _full_spec((1, 128)),
    ],
    out_specs=_row_spec(128),
    out_shape=jax.ShapeDtypeStruct((NPAD, 128), jnp.float32),
)


def kernel(x, edge_index, W1, b1, W2, b2, Wlin, blin):
    f32 = jnp.float32
    ppt = EPT - E // NW
    pad_dst = N + (jnp.arange(NW * ppt, dtype=jnp.int32) % NPR).reshape(NW, ppt)
    srcp = jnp.concatenate(
        [edge_index[0].reshape(NW, E // NW),
         jnp.zeros((NW, ppt), jnp.int32)], axis=1).reshape(-1)
    dstp = jnp.concatenate(
        [edge_index[1].reshape(NW, E // NW), pad_dst], axis=1).reshape(-1)

    xpad = jnp.pad(x, ((0, NPAD - N), (0, 0)))
    ones_c = jnp.ones((C,), f32)
    zeros_1d = jnp.zeros((RPT,), f32)
    zeros_rows = jnp.zeros((WB, IN), f32)

    degp = _deg_kernel(dstp, ones_c, zeros_1d)
    pt = degp.T
    xs = _xs_call(pt, xpad)
    s1 = _agg_kernel(srcp, dstp, xs, zeros_rows)
    y, ys = _mid_call(pt, xpad, s1, W1, b1.reshape(1, HID), W2)
    s2 = _agg_kernel(srcp, dstp, ys, zeros_rows)
    Wl = jnp.zeros((OUT, 128), f32).at[:, :2].set(Wlin)
    bl = jnp.zeros((1, 128), f32).at[0, :2].set(blin)
    o = _out_call(pt, y, s2, b2.reshape(1, OUT), Wl, bl)
    return o[:N, :2]


# speedup vs baseline: 1.5790x; 1.5790x over previous
import functools

import jax
import jax.numpy as jnp
from jax import lax
from jax.experimental import pallas as pl
from jax.experimental.pallas import tpu as pltpu
from jax.experimental.pallas import tpu_sc as plsc

N = 10000
NPAD = 10240
E = 320000
IN = 128
HID = 256
OUT = 128

NC = 2
NS = 16
NW = NC * NS
C = 80
CHPT = 125
EPT = CHPT * C
EPAD = NW * EPT
NPR = NPAD - N
RPT = NPAD // NS
APT = RPT
AWB = [(i * 80, 80) for i in range(8)]
WB = 80

_mesh = plsc.VectorSubcoreMesh(core_axis_name="c", subcore_axis_name="s")


@functools.partial(
    pl.kernel,
    out_type=jax.ShapeDtypeStruct((NC, NPAD), jnp.float32),
    mesh=_mesh,
    scratch_types=[
        [pltpu.VMEM((C,), jnp.int32)] * 4,
        pltpu.VMEM((C,), jnp.float32),
        pltpu.VMEM((RPT,), jnp.float32),
        pltpu.VMEM_SHARED((NPAD,), jnp.float32),
        [pltpu.SemaphoreType.DMA] * 4,
        [pltpu.SemaphoreType.DMA] * 4,
    ],
)
def _deg_kernel(dst_hbm, ones_hbm, zeros_hbm, out_hbm,
                idx, ones_v, stage_v, acc, semi, sems):
    c = lax.axis_index("c")
    s = lax.axis_index("s")
    pltpu.sync_copy(ones_hbm, ones_v)
    pltpu.sync_copy(zeros_hbm, stage_v)
    pltpu.sync_copy(stage_v, acc.at[pl.ds(s * RPT, RPT)])
    plsc.subcore_barrier()

    ebase = (c * NS + s) * EPT

    def d_at(j):
        return dst_hbm.at[pl.ds(pl.multiple_of(ebase + j * C, 8), C)]

    def load(j, b):
        pltpu.async_copy(d_at(j), idx[b], semi[b])

    def wait_scatter(b):
        pltpu.make_async_copy(ones_v, acc.at[idx[b]], sems[b]).wait()

    def step(j, b, do_load, do_wait):
        if do_load:
            if do_wait:
                wait_scatter((b + 2) % 4)
            load(j + 2, (b + 2) % 4)
        pltpu.make_async_copy(d_at(j), idx[b], semi[b]).wait()
        pltpu.async_copy(ones_v, acc.at[idx[b]], sems[b], add=True)

    load(0, 0)
    load(1, 1)

    def body(k, carry):
        for r in range(4):
            step(4 * k + 2 + r, (2 + r) % 4, True, True)
        return carry

    KMAIN = (CHPT - 4) // 4
    step(0, 0, True, False)
    step(1, 1, True, False)
    lax.fori_loop(0, KMAIN, body, 0)
    for j in range(4 * KMAIN + 2, CHPT):
        step(j, j % 4, j + 2 < CHPT, j - 2 >= 0)
    for j in range(max(0, CHPT - 4), CHPT):
        wait_scatter(j % 4)

    plsc.subcore_barrier()
    pltpu.sync_copy(acc.at[pl.ds(s * RPT, RPT)], stage_v)
    pltpu.sync_copy(stage_v, out_hbm.at[c, pl.ds(s * RPT, RPT)])


@functools.partial(
    pl.kernel,
    out_type=jax.ShapeDtypeStruct((NC, NPAD, IN), jnp.float32),
    mesh=_mesh,
    scratch_types=[
        [pltpu.VMEM((C,), jnp.int32)] * 4,
        [pltpu.VMEM((C,), jnp.int32)] * 4,
        [pltpu.VMEM((C, IN), jnp.float32)] * 4,
        pltpu.VMEM_SHARED((NPAD, IN), jnp.float32),
        [pltpu.SemaphoreType.DMA] * 4,
        [pltpu.SemaphoreType.DMA] * 4,
        [pltpu.SemaphoreType.DMA] * 4,
    ],
)
def _agg_kernel(src_hbm, dst_hbm, tab_hbm, zrows_hbm, out_hbm,
                srcv, dstv, rows, acc, semi, semg, sems):
    c = lax.axis_index("c")
    s = lax.axis_index("s")

    pltpu.sync_copy(zrows_hbm, rows[0])
    for r0, w in AWB:
        pltpu.sync_copy(rows[0].at[pl.ds(0, w)],
                        acc.at[pl.ds(s * APT + r0, w)])
    plsc.subcore_barrier()

    ebase = (c * NS + s) * EPT

    def s_at(j):
        return src_hbm.at[pl.ds(pl.multiple_of(ebase + j * C, 8), C)]

    def d_at(j):
        return dst_hbm.at[pl.ds(pl.multiple_of(ebase + j * C, 8), C)]

    def load_idx(j, b):
        pltpu.async_copy(s_at(j), srcv[b], semi[b])
        pltpu.async_copy(d_at(j), dstv[b], semi[b])

    def start_gather(j, b):
        pltpu.make_async_copy(s_at(j), srcv[b], semi[b]).wait()
        pltpu.make_async_copy(d_at(j), dstv[b], semi[b]).wait()
        pltpu.async_copy(tab_hbm.at[srcv[b]], rows[b], semg[b])

    def start_scatter(b):
        pltpu.make_async_copy(tab_hbm.at[srcv[b]], rows[b], semg[b]).wait()
        pltpu.async_copy(rows[b], acc.at[dstv[b]], sems[b], add=True)

    def wait_scatter(b):
        pltpu.make_async_copy(rows[b], acc.at[dstv[b]], sems[b]).wait()

    def step(j, b, do_load, do_wait, do_gather):
        if do_load:
            if do_wait:
                wait_scatter((b + 2) % 4)
            load_idx(j + 2, (b + 2) % 4)
        if do_gather:
            start_gather(j + 1, (b + 1) % 4)
        start_scatter(b)

    load_idx(0, 0)
    load_idx(1, 1)
    start_gather(0, 0)

    def body(k, carry):
        for r in range(4):
            step(4 * k + 2 + r, (2 + r) % 4, True, True, True)
        return carry

    KMAIN = (CHPT - 4) // 4
    step(0, 0, True, False, True)
    step(1, 1, True, False, True)
    lax.fori_loop(0, KMAIN, body, 0)
    for j in range(4 * KMAIN + 2, CHPT):
        step(j, j % 4, j + 2 < CHPT, j - 2 >= 0, j + 1 < CHPT)
    for j in range(max(0, CHPT - 4), CHPT):
        wait_scatter(j % 4)

    plsc.subcore_barrier()
    for r0, w in AWB:
        a0 = s * APT + r0
        pltpu.sync_copy(acc.at[pl.ds(a0, w)], rows[0].at[pl.ds(0, w)])
        pltpu.sync_copy(rows[0].at[pl.ds(0, w)], out_hbm.at[c, pl.ds(a0, w)])


RB = 1024
GRID = NPAD // RB


def _dis_block(pt):
    return lax.rsqrt(1.0 + pt[:, 0:1] + pt[:, 1:2])


def _xs_body(pt_ref, x_ref, xs_ref):
    dis = _dis_block(pt_ref[...])
    xs_ref[...] = x_ref[...] * dis


def _mid_body(pt_ref, x_ref, s1_ref, W1_ref, b1_ref, W2_ref, y_ref, ys_ref):
    dis = _dis_block(pt_ref[...])
    agg = s1_ref[0] + s1_ref[1]
    u = dis * agg + (dis * dis) * x_ref[...]
    h = jnp.dot(u, W1_ref[...], preferred_element_type=jnp.float32) + b1_ref[...]
    h = jnp.maximum(h, 0.0)
    y = jnp.dot(h, W2_ref[...], preferred_element_type=jnp.float32)
    y_ref[...] = y
    ys_ref[...] = y * dis


def _out_body(pt_ref, y_ref, s2_ref, b2_ref, Wl_ref, bl_ref, o_ref):
    dis = _dis_block(pt_ref[...])
    agg = s2_ref[0] + s2_ref[1]
    h2 = jnp.maximum(dis * agg + (dis * dis) * y_ref[...] + b2_ref[...], 0.0)
    o_ref[...] = jnp.dot(h2, Wl_ref[...], preferred_element_type=jnp.float32) + bl_ref[...]


def _row_spec(width):
    return pl.BlockSpec((RB, width), lambda i: (i, 0))


def _part_spec(width):
    return pl.BlockSpec((NC, RB, width), lambda i: (0, i, 0))


def _full_spec(shape):
    return pl.BlockSpec(shape, lambda i: tuple(0 for _ in shape))


_xs_call = pl.pallas_call(
    _xs_body,
    grid=(GRID,),
    in_specs=[_row_spec(2), _row_spec(IN)],
    out_specs=_row_spec(IN),
    out_shape=jax.ShapeDtypeStruct((NPAD, IN), jnp.float32),
)

_mid_call = pl.pallas_call(
    _mid_body,
    grid=(GRID,),
    in_specs=[
        _row_spec(2),
        _row_spec(IN),
        _part_spec(IN),
        _full_spec((IN, HID)),
        _full_spec((1, HID)),
        _full_spec((HID, OUT)),
    ],
    out_specs=[_row_spec(OUT), _row_spec(OUT)],
    out_shape=[
        jax.ShapeDtypeStruct((NPAD, OUT), jnp.float32),
        jax.ShapeDtypeStruct((NPAD, OUT), jnp.float32),
    ],
)

_out_call = pl.pallas_call(
    _out_body,
    grid=(GRID,),
    in_specs=[
        _row_spec(2),
        _row_spec(OUT),
        _part_spec(OUT),
        _full_spec((1, OUT)),
        _full_spec((OUT, 128)),
        _full_spec((1, 128)),
    ],
    out_specs=_row_spec(128),
    out_shape=jax.ShapeDtypeStruct((NPAD, 128), jnp.float32),
)


def kernel(x, edge_index, W1, b1, W2, b2, Wlin, blin):
    f32 = jnp.float32
    ppt = EPT - E // NW
    pad_dst = N + (jnp.arange(NW * ppt, dtype=jnp.int32) % NPR).reshape(NW, ppt)
    srcp = jnp.concatenate(
        [edge_index[0].reshape(NW, E // NW),
         jnp.zeros((NW, ppt), jnp.int32)], axis=1).reshape(-1)
    dstp = jnp.concatenate(
        [edge_index[1].reshape(NW, E // NW), pad_dst], axis=1).reshape(-1)

    xpad = jnp.pad(x, ((0, NPAD - N), (0, 0)))
    ones_c = jnp.ones((C,), f32)
    zeros_1d = jnp.zeros((RPT,), f32)
    zeros_rows = jnp.zeros((WB, IN), f32)

    degp = _deg_kernel(dstp, ones_c, zeros_1d)
    pt = degp.T
    xs = _xs_call(pt, xpad)
    s1 = _agg_kernel(srcp, dstp, xs, zeros_rows)
    y, ys = _mid_call(pt, xpad, s1, W1, b1.reshape(1, HID), W2)
    s2 = _agg_kernel(srcp, dstp, ys, zeros_rows)
    Wl = jnp.zeros((OUT, 128), f32).at[:, :2].set(Wlin)
    bl = jnp.zeros((1, 128), f32).at[0, :2].set(blin)
    o = _out_call(pt, y, s2, b2.reshape(1, OUT), Wl, bl)
    return o[:N, :2]


# speedup vs baseline: 1.6197x; 1.0258x over previous
import functools

import jax
import jax.numpy as jnp
from jax import lax
from jax.experimental import pallas as pl
from jax.experimental.pallas import tpu as pltpu
from jax.experimental.pallas import tpu_sc as plsc

N = 10000
NPAD = 10240
E = 320000
IN = 128
HID = 256
OUT = 128

NC = 2
NS = 16
NW = NC * NS
C = 80
CHPT = 125
EPT = CHPT * C
EPAD = NW * EPT
NPR = NPAD - N
RPT = NPAD // NS
APT = RPT
AWB = [(i * 80, 80) for i in range(8)]
WB = 80

_mesh = plsc.VectorSubcoreMesh(core_axis_name="c", subcore_axis_name="s")


@functools.partial(
    pl.kernel,
    out_type=jax.ShapeDtypeStruct((NC, NPAD), jnp.float32),
    mesh=_mesh,
    scratch_types=[
        [pltpu.VMEM((C,), jnp.int32)] * 4,
        pltpu.VMEM((C,), jnp.float32),
        pltpu.VMEM((RPT,), jnp.float32),
        pltpu.VMEM_SHARED((NPAD,), jnp.float32),
        [pltpu.SemaphoreType.DMA] * 4,
        [pltpu.SemaphoreType.DMA] * 4,
    ],
)
def _deg_kernel(dst_hbm, ones_hbm, zeros_hbm, out_hbm,
                idx, ones_v, stage_v, acc, semi, sems):
    c = lax.axis_index("c")
    s = lax.axis_index("s")
    pltpu.sync_copy(ones_hbm, ones_v)
    pltpu.sync_copy(zeros_hbm, stage_v)
    pltpu.sync_copy(stage_v, acc.at[pl.ds(s * RPT, RPT)])
    plsc.subcore_barrier()

    ebase = (c * NS + s) * EPT

    def d_at(j):
        return dst_hbm.at[pl.ds(pl.multiple_of(ebase + j * C, 8), C)]

    def load(j, b):
        pltpu.async_copy(d_at(j), idx[b], semi[b])

    def wait_scatter(b):
        pltpu.make_async_copy(ones_v, acc.at[idx[b]], sems[b]).wait()

    def step(j, b, do_load, do_wait):
        if do_load:
            if do_wait:
                wait_scatter((b + 2) % 4)
            load(j + 2, (b + 2) % 4)
        pltpu.make_async_copy(d_at(j), idx[b], semi[b]).wait()
        pltpu.async_copy(ones_v, acc.at[idx[b]], sems[b], add=True)

    load(0, 0)
    load(1, 1)

    def body(k, carry):
        for r in range(4):
            step(4 * k + 2 + r, (2 + r) % 4, True, True)
        return carry

    KMAIN = (CHPT - 4) // 4
    step(0, 0, True, False)
    step(1, 1, True, False)
    lax.fori_loop(0, KMAIN, body, 0)
    for j in range(4 * KMAIN + 2, CHPT):
        step(j, j % 4, j + 2 < CHPT, j - 2 >= 0)
    for j in range(max(0, CHPT - 4), CHPT):
        wait_scatter(j % 4)

    plsc.subcore_barrier()
    pltpu.sync_copy(acc.at[pl.ds(s * RPT, RPT)], stage_v)
    pltpu.sync_copy(stage_v, out_hbm.at[c, pl.ds(s * RPT, RPT)])


@functools.partial(
    pl.kernel,
    out_type=jax.ShapeDtypeStruct((NC, NPAD, IN), jnp.float32),
    mesh=_mesh,
    scratch_types=[
        [pltpu.VMEM((C,), jnp.int32)] * 4,
        [pltpu.VMEM((C,), jnp.int32)] * 4,
        [pltpu.VMEM((C, IN), jnp.float32)] * 4,
        pltpu.VMEM_SHARED((NPAD, IN), jnp.float32),
        [pltpu.SemaphoreType.DMA] * 4,
        [pltpu.SemaphoreType.DMA] * 4,
        [pltpu.SemaphoreType.DMA] * 4,
    ],
)
def _agg_kernel(src_hbm, dst_hbm, tab_hbm, zrows_hbm, out_hbm,
                srcv, dstv, rows, acc, semi, semg, sems):
    c = lax.axis_index("c")
    s = lax.axis_index("s")
    ebase = (c * NS + s) * EPT

    def s_at(j):
        return src_hbm.at[pl.ds(pl.multiple_of(ebase + j * C, 8), C)]

    def d_at(j):
        return dst_hbm.at[pl.ds(pl.multiple_of(ebase + j * C, 8), C)]

    def load_idx(j, b):
        pltpu.async_copy(s_at(j), srcv[b], semi[b])
        pltpu.async_copy(d_at(j), dstv[b], semi[b])

    def start_gather(j, b):
        pltpu.make_async_copy(s_at(j), srcv[b], semi[b]).wait()
        pltpu.make_async_copy(d_at(j), dstv[b], semi[b]).wait()
        pltpu.async_copy(tab_hbm.at[srcv[b]], rows[b], semg[b])

    def start_scatter(b):
        pltpu.make_async_copy(tab_hbm.at[srcv[b]], rows[b], semg[b]).wait()
        pltpu.async_copy(rows[b], acc.at[dstv[b]], sems[b], add=True)

    def wait_scatter(b):
        pltpu.make_async_copy(rows[b], acc.at[dstv[b]], sems[b]).wait()

    def step(j, b, do_load, do_wait, do_gather):
        if do_load:
            if do_wait:
                wait_scatter((b + 2) % 4)
            load_idx(j + 2, (b + 2) % 4)
        if do_gather:
            start_gather(j + 1, (b + 1) % 4)
        start_scatter(b)

    load_idx(0, 0)
    load_idx(1, 1)
    pltpu.sync_copy(zrows_hbm, rows[3])
    for r0, w in AWB:
        pltpu.async_copy(rows[3].at[pl.ds(0, w)],
                         acc.at[pl.ds(s * APT + r0, w)], sems[3])
    start_gather(0, 0)
    for r0, w in AWB:
        pltpu.make_async_copy(rows[3].at[pl.ds(0, w)],
                              acc.at[pl.ds(s * APT + r0, w)], sems[3]).wait()
    plsc.subcore_barrier()

    def body(k, carry):
        for r in range(4):
            step(4 * k + 2 + r, (2 + r) % 4, True, True, True)
        return carry

    KMAIN = (CHPT - 4) // 4
    step(0, 0, True, False, True)
    step(1, 1, True, False, True)
    lax.fori_loop(0, KMAIN, body, 0)
    for j in range(4 * KMAIN + 2, CHPT):
        step(j, j % 4, j + 2 < CHPT, j - 2 >= 0, j + 1 < CHPT)
    for j in range(max(0, CHPT - 4), CHPT):
        wait_scatter(j % 4)

    plsc.subcore_barrier()

    def wb_load(k):
        r0, w = AWB[k]
        pltpu.async_copy(acc.at[pl.ds(s * APT + r0, w)],
                         rows[k % 4].at[pl.ds(0, w)], semg[k % 4])

    def wb_store(k):
        r0, w = AWB[k]
        a0 = s * APT + r0
        pltpu.make_async_copy(acc.at[pl.ds(a0, w)],
                              rows[k % 4].at[pl.ds(0, w)], semg[k % 4]).wait()
        pltpu.async_copy(rows[k % 4].at[pl.ds(0, w)],
                         out_hbm.at[c, pl.ds(a0, w)], sems[k % 4])

    def wb_drain(k):
        r0, w = AWB[k]
        a0 = s * APT + r0
        pltpu.make_async_copy(rows[k % 4].at[pl.ds(0, w)],
                              out_hbm.at[c, pl.ds(a0, w)], sems[k % 4]).wait()

    nwb = len(AWB)
    for k in range(nwb):
        if k >= 4:
            wb_drain(k - 4)
        wb_load(k)
        if k >= 1:
            wb_store(k - 1)
    wb_store(nwb - 1)
    for k in range(max(0, nwb - 4), nwb):
        wb_drain(k)


RB = 1024
GRID = NPAD // RB


def _dis_block(pt):
    return lax.rsqrt(1.0 + pt[:, 0:1] + pt[:, 1:2])


def _xs_body(pt_ref, x_ref, xs_ref):
    dis = _dis_block(pt_ref[...])
    xs_ref[...] = x_ref[...] * dis


def _mid_body(pt_ref, x_ref, s1_ref, W1_ref, b1_ref, W2_ref, y_ref, ys_ref):
    dis = _dis_block(pt_ref[...])
    agg = s1_ref[0] + s1_ref[1]
    u = dis * agg + (dis * dis) * x_ref[...]
    h = jnp.dot(u, W1_ref[...], preferred_element_type=jnp.float32) + b1_ref[...]
    h = jnp.maximum(h, 0.0)
    y = jnp.dot(h, W2_ref[...], preferred_element_type=jnp.float32)
    y_ref[...] = y
    ys_ref[...] = y * dis


def _out_body(pt_ref, y_ref, s2_ref, b2_ref, Wl_ref, bl_ref, o_ref):
    dis = _dis_block(pt_ref[...])
    agg = s2_ref[0] + s2_ref[1]
    h2 = jnp.maximum(dis * agg + (dis * dis) * y_ref[...] + b2_ref[...], 0.0)
    o_ref[...] = jnp.dot(h2, Wl_ref[...], preferred_element_type=jnp.float32) + bl_ref[...]


def _row_spec(width):
    return pl.BlockSpec((RB, width), lambda i: (i, 0))


def _part_spec(width):
    return pl.BlockSpec((NC, RB, width), lambda i: (0, i, 0))


def _full_spec(shape):
    return pl.BlockSpec(shape, lambda i: tuple(0 for _ in shape))


_xs_call = pl.pallas_call(
    _xs_body,
    grid=(GRID,),
    in_specs=[_row_spec(2), _row_spec(IN)],
    out_specs=_row_spec(IN),
    out_shape=jax.ShapeDtypeStruct((NPAD, IN), jnp.float32),
)

_mid_call = pl.pallas_call(
    _mid_body,
    grid=(GRID,),
    in_specs=[
        _row_spec(2),
        _row_spec(IN),
        _part_spec(IN),
        _full_spec((IN, HID)),
        _full_spec((1, HID)),
        _full_spec((HID, OUT)),
    ],
    out_specs=[_row_spec(OUT), _row_spec(OUT)],
    out_shape=[
        jax.ShapeDtypeStruct((NPAD, OUT), jnp.float32),
        jax.ShapeDtypeStruct((NPAD, OUT), jnp.float32),
    ],
)

_out_call = pl.pallas_call(
    _out_body,
    grid=(GRID,),
    in_specs=[
        _row_spec(2),
        _row_spec(OUT),
        _part_spec(OUT),
        _full_spec((1, OUT)),
        _full_spec((OUT, 128)),
        _full_spec((1, 128)),
    ],
    out_specs=_row_spec(128),
    out_shape=jax.ShapeDtypeStruct((NPAD, 128), jnp.float32),
)


def kernel(x, edge_index, W1, b1, W2, b2, Wlin, blin):
    f32 = jnp.float32
    ppt = EPT - E // NW
    pad_dst = N + (jnp.arange(NW * ppt, dtype=jnp.int32) % NPR).reshape(NW, ppt)
    srcp = jnp.concatenate(
        [edge_index[0].reshape(NW, E // NW),
         jnp.zeros((NW, ppt), jnp.int32)], axis=1).reshape(-1)
    dstp = jnp.concatenate(
        [edge_index[1].reshape(NW, E // NW), pad_dst], axis=1).reshape(-1)

    xpad = jnp.pad(x, ((0, NPAD - N), (0, 0)))
    ones_c = jnp.ones((C,), f32)
    zeros_1d = jnp.zeros((RPT,), f32)
    zeros_rows = jnp.zeros((WB, IN), f32)

    degp = _deg_kernel(dstp, ones_c, zeros_1d)
    pt = degp.T
    xs = _xs_call(pt, xpad)
    s1 = _agg_kernel(srcp, dstp, xs, zeros_rows)
    y, ys = _mid_call(pt, xpad, s1, W1, b1.reshape(1, HID), W2)
    s2 = _agg_kernel(srcp, dstp, ys, zeros_rows)
    Wl = jnp.zeros((OUT, 128), f32).at[:, :2].set(Wlin)
    bl = jnp.zeros((1, 128), f32).at[0, :2].set(blin)
    o = _out_call(pt, y, s2, b2.reshape(1, OUT), Wl, bl)
    return o[:N, :2]
